# Initial kernel scaffold; baseline (speedup 1.0000x reference)
import functools

import jax
import jax.numpy as jnp
from jax import lax
from jax.experimental import pallas as pl
from jax.experimental.pallas import tpu as pltpu

NUM_CONN = 32
LANES = 128


def _tree_reduce(vals, combine):
    while len(vals) > 1:
        nxt = [combine(vals[j], vals[j + 1]) for j in range(0, len(vals) - 1, 2)]
        if len(vals) % 2:
            nxt.append(vals[-1])
        vals = nxt
    return vals[0]


def _ddlg_body(conn_ref, order_ref, starts_ref, x_ref, o_ref):
    def segment(opcode, combine, leaf, fin):
        lo = starts_ref[opcode]
        hi = starts_ref[opcode + 1]

        def it(i, carry):
            rows = [leaf(x_ref[conn_ref[i, k]]) for k in range(NUM_CONN)]
            o_ref[order_ref[i]] = fin(_tree_reduce(rows, combine))
            return carry

        lax.fori_loop(lo, hi, it, 0)

    ident = lambda v: v
    inv = lambda v: 1.0 - v
    segment(0, jnp.minimum, ident, ident)
    segment(1, jnp.maximum, ident, ident)
    segment(2, lax.mul, ident, ident)
    segment(3, lax.mul, inv, inv)


@functools.partial(jax.jit, static_argnames=("out_f", "g"))
def _ddlg_layer(x3, conn_s, order, starts, out_f, g):
    in_f, nb, _ = x3.shape
    grid = (nb // g,)
    return pl.pallas_call(
        _ddlg_body,
        grid_spec=pltpu.PrefetchScalarGridSpec(
            num_scalar_prefetch=3,
            grid=grid,
            in_specs=[pl.BlockSpec((in_f, g, LANES), lambda i, *_: (0, i, 0))],
            out_specs=pl.BlockSpec((out_f, g, LANES), lambda i, *_: (0, i, 0)),
        ),
        out_shape=jax.ShapeDtypeStruct((out_f, nb, LANES), jnp.float32),
        compiler_params=pltpu.CompilerParams(
            dimension_semantics=("arbitrary",),
        ),
    )(conn_s, order, starts, x3)


def _sort_by_opcode(W, conn):
    opcode = jnp.argmax(W, axis=-1).astype(jnp.int32)
    order = jnp.argsort(opcode, stable=True).astype(jnp.int32)
    starts = jnp.searchsorted(
        opcode[order], jnp.arange(5, dtype=jnp.int32), side="left"
    ).astype(jnp.int32)
    conn_s = conn[order]
    return conn_s, order, starts


def kernel(x, W0, W1, W2, W3, conn0, conn1, conn2, conn3):
    Ws = [W0, W1, W2, W3]
    conns = [conn0, conn1, conn2, conn3]
    batch, in_f = x.shape
    nb = batch // LANES
    h = jnp.transpose(x).reshape(in_f, nb, LANES)
    for W, conn in zip(Ws, conns):
        conn_s, order, starts = _sort_by_opcode(W, conn)
        h = _ddlg_layer(h, conn_s, order, starts, out_f=W.shape[0], g=8)
    out_f = h.shape[0]
    return jnp.transpose(h.reshape(out_f, batch))


# speedup vs baseline: 4.2158x; 4.2158x over previous
import functools

import jax
import jax.numpy as jnp
from jax import lax
from jax.experimental import pallas as pl
from jax.experimental.pallas import tpu as pltpu

NUM_CONN = 32
LANES = 128


def _tree_reduce(vals, combine):
    while len(vals) > 1:
        nxt = [combine(vals[j], vals[j + 1]) for j in range(0, len(vals) - 1, 2)]
        if len(vals) % 2:
            nxt.append(vals[-1])
        vals = nxt
    return vals[0]


def _ddlg_body(conn_ref, order_ref, starts_ref, x_ref, o_ref):
    def segment(opcode, combine, leaf, fin):
        lo = starts_ref[opcode]
        hi = starts_ref[opcode + 1]

        def it(i, carry):
            base = i * NUM_CONN
            rows = [leaf(x_ref[conn_ref[base + k]]) for k in range(NUM_CONN)]
            o_ref[order_ref[i]] = fin(_tree_reduce(rows, combine))
            return carry

        lax.fori_loop(lo, hi, it, 0)

    ident = lambda v: v
    inv = lambda v: 1.0 - v
    segment(0, jnp.minimum, ident, ident)
    segment(1, jnp.maximum, ident, ident)
    segment(2, lax.mul, ident, ident)
    segment(3, lax.mul, inv, inv)


@functools.partial(jax.jit, static_argnames=("out_f", "g"))
def _ddlg_layer(x3, conn_s, order, starts, out_f, g):
    in_f, nb, _ = x3.shape
    grid = (nb // g,)
    return pl.pallas_call(
        _ddlg_body,
        grid_spec=pltpu.PrefetchScalarGridSpec(
            num_scalar_prefetch=3,
            grid=grid,
            in_specs=[pl.BlockSpec((in_f, g, LANES), lambda i, *_: (0, i, 0))],
            out_specs=pl.BlockSpec((out_f, g, LANES), lambda i, *_: (0, i, 0)),
        ),
        out_shape=jax.ShapeDtypeStruct((out_f, nb, LANES), jnp.float32),
        compiler_params=pltpu.CompilerParams(
            dimension_semantics=("arbitrary",),
        ),
    )(conn_s, order, starts, x3)


def _sort_by_opcode(W, conn):
    opcode = jnp.argmax(W, axis=-1).astype(jnp.int32)
    order = jnp.argsort(opcode, stable=True).astype(jnp.int32)
    starts = jnp.searchsorted(
        opcode[order], jnp.arange(5, dtype=jnp.int32), side="left"
    ).astype(jnp.int32)
    conn_s = conn[order].reshape(-1)
    return conn_s, order, starts


def kernel(x, W0, W1, W2, W3, conn0, conn1, conn2, conn3):
    Ws = [W0, W1, W2, W3]
    conns = [conn0, conn1, conn2, conn3]
    batch, in_f = x.shape
    nb = batch // LANES
    h = jnp.transpose(x).reshape(in_f, nb, LANES)
    for W, conn in zip(Ws, conns):
        conn_s, order, starts = _sort_by_opcode(W, conn)
        h = _ddlg_layer(h, conn_s, order, starts, out_f=W.shape[0], g=min(8, nb))
    out_f = h.shape[0]
    return jnp.transpose(h.reshape(out_f, batch))


# speedup vs baseline: 8.1947x; 1.9438x over previous
import functools

import jax
import jax.numpy as jnp
from jax import lax
from jax.experimental import pallas as pl
from jax.experimental.pallas import tpu as pltpu

NUM_CONN = 32
LANES = 128


def _tree_reduce(vals, combine):
    while len(vals) > 1:
        nxt = [combine(vals[j], vals[j + 1]) for j in range(0, len(vals) - 1, 2)]
        if len(vals) % 2:
            nxt.append(vals[-1])
        vals = nxt
    return vals[0]


def _ddlg_body(conn_ref, order_ref, starts_ref, x_ref, o_ref, *, half):
    j = pl.program_id(1)

    def segment(opcode, combine, leaf, fin):
        lo = starts_ref[j, opcode]
        hi = starts_ref[j, opcode + 1]

        def it(i, carry):
            base = i * NUM_CONN
            rows = [leaf(x_ref[conn_ref[base + k]]) for k in range(NUM_CONN)]
            o_ref[order_ref[i] - j * half] = fin(_tree_reduce(rows, combine))
            return carry

        lax.fori_loop(lo, hi, it, 0)

    ident = lambda v: v
    inv = lambda v: 1.0 - v
    segment(0, jnp.minimum, ident, ident)
    segment(1, jnp.maximum, ident, ident)
    segment(2, lax.mul, ident, ident)
    segment(3, lax.mul, inv, inv)


@functools.partial(jax.jit, static_argnames=("out_f", "g", "out_split"))
def _ddlg_layer(x3, conn_s, order, starts, out_f, g, out_split):
    in_f, nb, _ = x3.shape
    half = out_f // out_split
    grid = (nb // g, out_split)
    return pl.pallas_call(
        functools.partial(_ddlg_body, half=half),
        grid_spec=pltpu.PrefetchScalarGridSpec(
            num_scalar_prefetch=3,
            grid=grid,
            in_specs=[pl.BlockSpec((in_f, g, LANES), lambda i, j, *_: (0, i, 0))],
            out_specs=pl.BlockSpec((half, g, LANES), lambda i, j, *_: (j, i, 0)),
        ),
        out_shape=jax.ShapeDtypeStruct((out_f, nb, LANES), jnp.float32),
        compiler_params=pltpu.CompilerParams(
            dimension_semantics=("arbitrary", "arbitrary"),
        ),
    )(conn_s, order, starts, x3)


def _sort_by_opcode(W, conn, out_split):
    opcode = jnp.argmax(W, axis=-1).astype(jnp.int32)
    out_f = W.shape[0]
    half = out_f // out_split
    rows = jnp.arange(out_f, dtype=jnp.int32)
    key = (rows // half) * 4 + opcode
    order = jnp.argsort(key, stable=True).astype(jnp.int32)
    bounds = jnp.searchsorted(
        key[order], jnp.arange(out_split * 4 + 1, dtype=jnp.int32), side="left"
    ).astype(jnp.int32)
    starts2 = bounds[jnp.arange(out_split)[:, None] * 4
                     + jnp.arange(5)[None, :]]
    conn_s = conn[order].reshape(-1)
    return conn_s, order, starts2


def kernel(x, W0, W1, W2, W3, conn0, conn1, conn2, conn3):
    Ws = [W0, W1, W2, W3]
    conns = [conn0, conn1, conn2, conn3]
    batch, in_f = x.shape
    nb = batch // LANES
    h = jnp.transpose(x).reshape(in_f, nb, LANES)
    for W, conn in zip(Ws, conns):
        out_split = 2
        conn_s, order, starts = _sort_by_opcode(W, conn, out_split)
        h = _ddlg_layer(h, conn_s, order, starts, out_f=W.shape[0],
                        g=min(16, nb), out_split=out_split)
    out_f = h.shape[0]
    return jnp.transpose(h.reshape(out_f, batch))


# speedup vs baseline: 12.9250x; 1.5772x over previous
import functools

import jax
import jax.numpy as jnp
from jax import lax
from jax.experimental import pallas as pl
from jax.experimental.pallas import tpu as pltpu
from jax.experimental.pallas import tpu_sc as plsc

NUM_CONN = 32
LANES = 128

SC_COLS = 8192
SC_NC = 2
SC_NS = 16
SC_CC = 32
SC_OB = 256


def _tree_reduce(vals, combine):
    while len(vals) > 1:
        nxt = [combine(vals[j], vals[j + 1]) for j in range(0, len(vals) - 1, 2)]
        if len(vals) % 2:
            nxt.append(vals[-1])
        vals = nxt
    return vals[0]


def _ddlg_body(conn_ref, order_ref, starts_ref, x_hbm, o_ref, x_ref, sem,
               *, half, g):
    i = pl.program_id(0)
    j = pl.program_id(1)

    @pl.when(j == 0)
    def _():
        cp = pltpu.make_async_copy(
            x_hbm.at[:, pl.ds(i * g, g), :], x_ref, sem)
        cp.start()
        cp.wait()

    def segment(opcode, combine, leaf, fin):
        lo = starts_ref[j, opcode]
        hi = starts_ref[j, opcode + 1]

        def it(i, carry):
            base = i * NUM_CONN
            rows = [leaf(x_ref[conn_ref[base + k]]) for k in range(NUM_CONN)]
            o_ref[order_ref[i] - j * half] = fin(_tree_reduce(rows, combine))
            return carry

        lax.fori_loop(lo, hi, it, 0)

    ident = lambda v: v
    inv = lambda v: 1.0 - v
    segment(0, jnp.minimum, ident, ident)
    segment(1, jnp.maximum, ident, ident)
    segment(2, lax.mul, ident, ident)
    segment(3, lax.mul, inv, inv)


@functools.partial(jax.jit, static_argnames=("out_f", "g", "out_split"))
def _ddlg_layer(x3, conn_s, order, starts, out_f, g, out_split):
    in_f, nb, _ = x3.shape
    half = out_f // out_split
    grid = (nb // g, out_split)
    return pl.pallas_call(
        functools.partial(_ddlg_body, half=half, g=g),
        grid_spec=pltpu.PrefetchScalarGridSpec(
            num_scalar_prefetch=3,
            grid=grid,
            in_specs=[pl.BlockSpec(memory_space=pltpu.MemorySpace.HBM)],
            out_specs=pl.BlockSpec((half, g, LANES), lambda i, j, *_: (j, i, 0)),
            scratch_shapes=[
                pltpu.VMEM((in_f, g, LANES), jnp.float32),
                pltpu.SemaphoreType.DMA,
            ],
        ),
        out_shape=jax.ShapeDtypeStruct((out_f, nb, LANES), jnp.float32),
        compiler_params=pltpu.CompilerParams(
            dimension_semantics=("arbitrary", "arbitrary"),
        ),
    )(conn_s, order, starts, x3)


def _sort_by_opcode(W, conn, out_split):
    opcode = jnp.argmax(W, axis=-1).astype(jnp.int32)
    out_f = W.shape[0]
    half = out_f // out_split
    rows = jnp.arange(out_f, dtype=jnp.int32)
    key = (rows // half) * 4 + opcode
    order = jnp.argsort(key, stable=True).astype(jnp.int32)
    bounds = jnp.searchsorted(
        key[order], jnp.arange(out_split * 4 + 1, dtype=jnp.int32), side="left"
    ).astype(jnp.int32)
    starts2 = bounds[jnp.arange(out_split)[:, None] * 4
                     + jnp.arange(5)[None, :]]
    conn_s = conn[order].reshape(-1)
    return conn_s, order, starts2


def kernel(x, W0, W1, W2, W3, conn0, conn1, conn2, conn3):
    Ws = [W0, W1, W2, W3]
    conns = [conn0, conn1, conn2, conn3]
    batch, in_f = x.shape
    nb = batch // LANES
    h = jnp.transpose(x).reshape(in_f, nb, LANES)
    for W, conn in zip(Ws, conns):
        out_split = 8
        conn_s, order, starts = _sort_by_opcode(W, conn, out_split)
        h = _ddlg_layer(h, conn_s, order, starts, out_f=W.shape[0],
                        g=min(32, nb), out_split=out_split)
    out_f = h.shape[0]
    return jnp.transpose(h.reshape(out_f, batch))
